# baseline (speedup 1.0000x reference)
import functools

import jax
import jax.numpy as jnp
from jax import lax
from jax.experimental import pallas as pl
from jax.experimental.pallas import tpu as pltpu
from jax.experimental.pallas import tpu_sc as plsc

N_NODES = 10000
N_EDGES = 160000
D_IN = 256
D_OUT = 256
HALF = 128

NUM_CORES = 2
NUM_TILES = 16
CHUNK = 80
K = 128
NCHUNKS = NUM_TILES * K
E_PAD = NCHUNKS * CHUNK
ROWS_PER_TILE = 632
N_PAD = NUM_TILES * ROWS_PER_TILE
NBUF = 4
NPH = 4
HK = K // NPH


def _sc_body(table, src2, dst2, out, sidx, didx, b0, b1, b2, b3, acc,
             g0, g1, g2, g3, c0, c1, c2, c3):
    c = lax.axis_index("c")
    s = lax.axis_index("s")
    bufs = (b0, b1, b2, b3)
    gsems = (g0, g1, g2, g3)
    csems = (c0, c1, c2, c3)

    def zrow(r, _):
        for j in range(HALF // 16):
            b0[r, pl.ds(j * 16, 16)] = jnp.zeros((16,), jnp.float32)
        return 0
    lax.fori_loop(0, CHUNK, zrow, 0)

    for off in range(0, ROWS_PER_TILE - CHUNK, CHUNK):
        pltpu.sync_copy(b0, acc.at[pl.ds(s * ROWS_PER_TILE + off, CHUNK)])
    pltpu.sync_copy(b0.at[pl.ds(0, ROWS_PER_TILE % CHUNK)],
                    acc.at[pl.ds(s * ROWS_PER_TILE
                                 + ROWS_PER_TILE - ROWS_PER_TILE % CHUNK,
                                 ROWS_PER_TILE % CHUNK)])

    plsc.subcore_barrier()

    for p in range(NPH):
        pltpu.sync_copy(src2.at[c, pl.ds(s * K + p * HK, HK)], sidx)
        pltpu.sync_copy(dst2.at[pl.ds(s * K + p * HK, HK)], didx)

        for b in range(NBUF):
            pltpu.async_copy(table.at[sidx.at[b]], bufs[b], gsems[b])

        def body(t, _):
            for b in range(NBUF):
                g = NBUF * t + b
                pltpu.make_async_copy(table.at[sidx.at[g]], bufs[b],
                                      gsems[b]).wait()
                pltpu.async_copy(bufs[b], acc.at[didx.at[g]], csems[b],
                                 add=True)

                @pl.when(g + NBUF < HK)
                def _(b=b, g=g):
                    pltpu.make_async_copy(bufs[b], acc.at[didx.at[g]],
                                          csems[b]).wait()
                    pltpu.async_copy(table.at[sidx.at[g + NBUF]], bufs[b],
                                     gsems[b])
            return 0
        lax.fori_loop(0, HK // NBUF, body, 0)

        for b in range(NBUF):
            pltpu.make_async_copy(bufs[b], acc.at[didx.at[HK - NBUF + b]],
                                  csems[b]).wait()

    plsc.subcore_barrier()

    pltpu.sync_copy(acc.at[pl.ds(s * ROWS_PER_TILE, ROWS_PER_TILE)],
                    out.at[c, pl.ds(s * ROWS_PER_TILE, ROWS_PER_TILE)])


_sc_gather_scatter = functools.partial(
    pl.kernel,
    out_type=jax.ShapeDtypeStruct((NUM_CORES, N_PAD, HALF), jnp.float32),
    mesh=plsc.VectorSubcoreMesh(core_axis_name="c", subcore_axis_name="s"),
    scratch_types=[
        pltpu.VMEM((HK, CHUNK), jnp.int32),
        pltpu.VMEM((HK, CHUNK), jnp.int32),
        pltpu.VMEM((CHUNK, HALF), jnp.float32),
        pltpu.VMEM((CHUNK, HALF), jnp.float32),
        pltpu.VMEM((CHUNK, HALF), jnp.float32),
        pltpu.VMEM((CHUNK, HALF), jnp.float32),
        pltpu.VMEM_SHARED((N_PAD, HALF), jnp.float32),
        pltpu.SemaphoreType.DMA,
        pltpu.SemaphoreType.DMA,
        pltpu.SemaphoreType.DMA,
        pltpu.SemaphoreType.DMA,
        pltpu.SemaphoreType.DMA,
        pltpu.SemaphoreType.DMA,
        pltpu.SemaphoreType.DMA,
        pltpu.SemaphoreType.DMA,
    ],
)(_sc_body)


def _tc_body(hn_ref, wt_ref, b_ref, o_ref):
    o_ref[...] = (
        jnp.dot(hn_ref[0], wt_ref[0], preferred_element_type=jnp.float32)
        + jnp.dot(hn_ref[1], wt_ref[1], preferred_element_type=jnp.float32)
        + b_ref[...]
    )


def _tc_linear(hn, wt2, b2):
    bm = 512
    return pl.pallas_call(
        _tc_body,
        grid=(pl.cdiv(N_NODES, bm),),
        in_specs=[
            pl.BlockSpec((NUM_CORES, bm, HALF), lambda i: (0, i, 0)),
            pl.BlockSpec((NUM_CORES, HALF, D_OUT), lambda i: (0, 0, 0)),
            pl.BlockSpec((1, D_OUT), lambda i: (0, 0)),
        ],
        out_specs=pl.BlockSpec((bm, D_OUT), lambda i: (i, 0)),
        out_shape=jax.ShapeDtypeStruct((N_NODES, D_OUT), jnp.float32),
    )(hn, wt2, b2)


def kernel(h, edge_index, W, b):
    table = h.reshape(N_NODES, NUM_CORES, HALF).transpose(1, 0, 2).reshape(
        NUM_CORES * N_NODES, HALF)

    src = edge_index[0].astype(jnp.int32)
    dst = edge_index[1].astype(jnp.int32)
    pad = E_PAD - N_EDGES
    src_p = jnp.concatenate([src, jnp.zeros((pad,), jnp.int32)])
    dst_p = jnp.concatenate(
        [dst, N_NODES + (jnp.arange(pad, dtype=jnp.int32) % (N_PAD - N_NODES))])
    src2 = jnp.stack([src_p, src_p + N_NODES]).reshape(
        NUM_CORES, NCHUNKS, CHUNK)
    dst2 = dst_p.reshape(NCHUNKS, CHUNK)

    hn = _sc_gather_scatter(table, src2, dst2)

    wt2 = W.T.reshape(NUM_CORES, HALF, D_OUT)
    return _tc_linear(hn, wt2, b.reshape(1, D_OUT))


# speedup vs baseline: 1.0097x; 1.0097x over previous
import functools

import jax
import jax.numpy as jnp
from jax import lax
from jax.experimental import pallas as pl
from jax.experimental.pallas import tpu as pltpu
from jax.experimental.pallas import tpu_sc as plsc

N_NODES = 10000
N_EDGES = 160000
D_IN = 256
D_OUT = 256
HALF = 128

NUM_CORES = 2
NUM_TILES = 16
CHUNK = 64
K = 160
NCHUNKS = NUM_TILES * K
E_PAD = NCHUNKS * CHUNK
ROWS_PER_TILE = 632
N_PAD = NUM_TILES * ROWS_PER_TILE
NBUF = 4
NPH = 4
HK = K // NPH


def _sc_body(table, src2, dst2, out, sidx, didx, b0, b1, b2, b3, acc,
             g0, g1, g2, g3, c0, c1, c2, c3):
    c = lax.axis_index("c")
    s = lax.axis_index("s")
    bufs = (b0, b1, b2, b3)
    gsems = (g0, g1, g2, g3)
    csems = (c0, c1, c2, c3)

    def zrow(r, _):
        for j in range(HALF // 16):
            b0[r, pl.ds(j * 16, 16)] = jnp.zeros((16,), jnp.float32)
        return 0
    lax.fori_loop(0, CHUNK, zrow, 0)

    for off in range(0, ROWS_PER_TILE - CHUNK, CHUNK):
        pltpu.sync_copy(b0, acc.at[pl.ds(s * ROWS_PER_TILE + off, CHUNK)])
    pltpu.sync_copy(b0.at[pl.ds(0, ROWS_PER_TILE % CHUNK)],
                    acc.at[pl.ds(s * ROWS_PER_TILE
                                 + ROWS_PER_TILE - ROWS_PER_TILE % CHUNK,
                                 ROWS_PER_TILE % CHUNK)])

    plsc.subcore_barrier()

    for p in range(NPH):
        pltpu.sync_copy(src2.at[c, pl.ds(s * K + p * HK, HK)], sidx)
        pltpu.sync_copy(dst2.at[pl.ds(s * K + p * HK, HK)], didx)

        for b in range(NBUF):
            pltpu.async_copy(table.at[sidx.at[b]], bufs[b], gsems[b])

        def body(t, _):
            for b in range(NBUF):
                g = NBUF * t + b
                pltpu.make_async_copy(table.at[sidx.at[g]], bufs[b],
                                      gsems[b]).wait()
                pltpu.async_copy(bufs[b], acc.at[didx.at[g]], csems[b],
                                 add=True)

                @pl.when(g + NBUF < HK)
                def _(b=b, g=g):
                    pltpu.make_async_copy(bufs[b], acc.at[didx.at[g]],
                                          csems[b]).wait()
                    pltpu.async_copy(table.at[sidx.at[g + NBUF]], bufs[b],
                                     gsems[b])
            return 0
        lax.fori_loop(0, HK // NBUF, body, 0)

        for b in range(NBUF):
            pltpu.make_async_copy(bufs[b], acc.at[didx.at[HK - NBUF + b]],
                                  csems[b]).wait()

    plsc.subcore_barrier()

    pltpu.sync_copy(acc.at[pl.ds(s * ROWS_PER_TILE, ROWS_PER_TILE)],
                    out.at[c, pl.ds(s * ROWS_PER_TILE, ROWS_PER_TILE)])


_sc_gather_scatter = functools.partial(
    pl.kernel,
    out_type=jax.ShapeDtypeStruct((NUM_CORES, N_PAD, HALF), jnp.float32),
    mesh=plsc.VectorSubcoreMesh(core_axis_name="c", subcore_axis_name="s"),
    scratch_types=[
        pltpu.VMEM((HK, CHUNK), jnp.int32),
        pltpu.VMEM((HK, CHUNK), jnp.int32),
        pltpu.VMEM((CHUNK, HALF), jnp.float32),
        pltpu.VMEM((CHUNK, HALF), jnp.float32),
        pltpu.VMEM((CHUNK, HALF), jnp.float32),
        pltpu.VMEM((CHUNK, HALF), jnp.float32),
        pltpu.VMEM_SHARED((N_PAD, HALF), jnp.float32),
        pltpu.SemaphoreType.DMA,
        pltpu.SemaphoreType.DMA,
        pltpu.SemaphoreType.DMA,
        pltpu.SemaphoreType.DMA,
        pltpu.SemaphoreType.DMA,
        pltpu.SemaphoreType.DMA,
        pltpu.SemaphoreType.DMA,
        pltpu.SemaphoreType.DMA,
    ],
)(_sc_body)


def _tc_body(hn_ref, wt_ref, b_ref, o_ref):
    o_ref[...] = (
        jnp.dot(hn_ref[0], wt_ref[0], preferred_element_type=jnp.float32)
        + jnp.dot(hn_ref[1], wt_ref[1], preferred_element_type=jnp.float32)
        + b_ref[...]
    )


def _tc_linear(hn, wt2, b2):
    bm = 512
    return pl.pallas_call(
        _tc_body,
        grid=(pl.cdiv(N_NODES, bm),),
        in_specs=[
            pl.BlockSpec((NUM_CORES, bm, HALF), lambda i: (0, i, 0)),
            pl.BlockSpec((NUM_CORES, HALF, D_OUT), lambda i: (0, 0, 0)),
            pl.BlockSpec((1, D_OUT), lambda i: (0, 0)),
        ],
        out_specs=pl.BlockSpec((bm, D_OUT), lambda i: (i, 0)),
        out_shape=jax.ShapeDtypeStruct((N_NODES, D_OUT), jnp.float32),
    )(hn, wt2, b2)


def kernel(h, edge_index, W, b):
    table = h.reshape(N_NODES, NUM_CORES, HALF).transpose(1, 0, 2).reshape(
        NUM_CORES * N_NODES, HALF)

    src = edge_index[0].astype(jnp.int32)
    dst = edge_index[1].astype(jnp.int32)
    pad = E_PAD - N_EDGES
    src_p = jnp.concatenate([src, jnp.zeros((pad,), jnp.int32)])
    dst_p = jnp.concatenate(
        [dst, N_NODES + (jnp.arange(pad, dtype=jnp.int32) % (N_PAD - N_NODES))])
    src2 = jnp.stack([src_p, src_p + N_NODES]).reshape(
        NUM_CORES, NCHUNKS, CHUNK)
    dst2 = dst_p.reshape(NCHUNKS, CHUNK)

    hn = _sc_gather_scatter(table, src2, dst2)

    wt2 = W.T.reshape(NUM_CORES, HALF, D_OUT)
    return _tc_linear(hn, wt2, b.reshape(1, D_OUT))
